# baseline (speedup 1.0000x reference)
import functools

import jax
import jax.numpy as jnp
from jax import lax
from jax.experimental import pallas as pl
from jax.experimental.pallas import tpu as pltpu
from jax.experimental.pallas import tpu_sc as plsc

_MAP_H = 64
_MAP_W = 128
_L1_REG = 0.1
_WEIGHT = 1.0


def _make_sc_gather(B, N, T, H, W):
    info = plsc.get_sparse_core_info()
    NC, NS, L = info.num_cores, info.num_subcores, info.num_lanes
    NW = NC * NS
    total_traj = B * N
    traj_per_w = total_traj // NW
    assert total_traj % NW == 0 and traj_per_w % 8 == 0
    n_full = T // L
    tail = T - n_full * L
    mesh = plsc.VectorSubcoreMesh(core_axis_name="c", subcore_axis_name="s")

    @functools.partial(
        pl.kernel,
        mesh=mesh,
        compiler_params=pltpu.CompilerParams(needs_layout_passes=False),
        out_type=jax.ShapeDtypeStruct((total_traj,), jnp.float32),
        scratch_types=[
            pltpu.VMEM((H * W,), jnp.float32),
            pltpu.VMEM((traj_per_w * T * 2,), jnp.int32),
            pltpu.VMEM((traj_per_w,), jnp.float32),
            pltpu.SemaphoreType.DMA,
        ],
    )
    def sc_gather(pred_hbm, poses_hbm, out_hbm,
                  pred_v, poses_v, reward_v, sem):
        wid = lax.axis_index("s") * NC + lax.axis_index("c")
        base = wid * traj_per_w
        b = base // N
        cp1 = pltpu.async_copy(pred_hbm.at[b], pred_v, sem)
        cp2 = pltpu.async_copy(
            poses_hbm.at[pl.ds(base * T * 2, traj_per_w * T * 2)], poses_v, sem)
        cp1.wait()
        cp2.wait()

        lane = lax.iota(jnp.int32, L)
        lane2 = lane * 2
        tail_mask = lane >= (L - tail)
        write_mask = lane == 0

        def body(n, carry):
            pair_base = n * (T * 2)
            acc = jnp.zeros((L,), jnp.float32)
            for j in range(n_full):
                off = lane2 + (pair_base + j * (2 * L))
                r = plsc.load_gather(poses_v, [off])
                c = plsc.load_gather(poses_v, [off + 1])
                acc = acc + plsc.load_gather(pred_v, [r * W + c])
            if tail:
                off = lane2 + (pair_base + (T - L) * 2)
                r = plsc.load_gather(poses_v, [off])
                c = plsc.load_gather(poses_v, [off + 1])
                g = plsc.load_gather(pred_v, [r * W + c])
                acc = acc + jnp.where(tail_mask, g, 0.0)
            total = jnp.sum(acc)
            plsc.store_scatter(reward_v, [jnp.full((L,), n, jnp.int32)],
                               jnp.full((L,), total, jnp.float32),
                               mask=write_mask)
            return carry

        lax.fori_loop(0, traj_per_w, body, 0)
        pltpu.sync_copy(reward_v, out_hbm.at[pl.ds(base, traj_per_w)])

    return sc_gather


def _tc_loss_body(pred_ref, reward_ref, ranks_ref, out_ref, *, B, N, n_elem):
    predv = pred_ref[...]
    l1 = jnp.sum(jnp.abs(predv)) / n_elem

    ranks = ranks_ref[...]
    reward = reward_ref[...]
    pref = ranks == 0
    nprf = ranks > 0
    preff = pref.astype(jnp.float32)
    nprff = nprf.astype(jnp.float32)

    ii = lax.broadcasted_iota(jnp.int32, (N, N), 0)
    jj = lax.broadcasted_iota(jnp.int32, (N, N), 1)
    tri = (ii <= jj).astype(jnp.float32)
    pos_p = lax.dot(preff, tri).astype(jnp.int32) - 1
    pos_q = lax.dot(nprff, tri).astype(jnp.int32) - 1

    Np = jnp.sum(preff, axis=1, keepdims=True)
    Nn = jnp.sum(nprff, axis=1, keepdims=True)
    Npi = Np.astype(jnp.int32)
    Nni = Nn.astype(jnp.int32)

    def gcd_step(_, xy):
        x, y = xy
        cont = y > 0
        return (jnp.where(cont, y, x),
                jnp.where(cont, lax.rem(x, jnp.maximum(y, 1)), 0))

    gi, _ = lax.fori_loop(0, 12, gcd_step, (Npi, Nni))
    gsafe = jnp.maximum(gi, 1)
    rp = lax.rem(jnp.maximum(pos_p, 0), gsafe)
    rq = lax.rem(jnp.maximum(pos_q, 0), gsafe)

    A = reward[:, :, None]
    C = reward[:, None, :]
    m2 = jnp.maximum(A, C)
    nc = m2 + jnp.log1p(jnp.exp(-jnp.abs(A - C)))
    ap = A - nc
    cp = C - nc
    p1 = ap / (ap + cp + 1e-6)

    eq = rp.astype(jnp.float32)[:, :, None] == rq.astype(jnp.float32)[:, None, :]
    wf = (preff[:, :, None] * nprff[:, None, :]) * jnp.where(eq, 1.0, 0.0)
    w = wf > 0
    gf = gi.astype(jnp.float32)[:, :, None]

    nv = (Np * Nn)[:, 0]
    has = nv > 0
    S = jnp.sum(wf * p1 * gf, axis=(1, 2))
    M = jnp.max(jnp.where(w, p1, -jnp.inf), axis=(1, 2))
    Msafe = jnp.where(has, M, 0.0)
    Z = jnp.sum(wf * jnp.exp(p1 - Msafe[:, None, None]), axis=(1, 2)) * gf[:, 0, 0]
    logZ = jnp.where(has, jnp.log(jnp.maximum(Z, 1e-30)), 0.0)
    cls = jnp.where(has, nv * (Msafe + logZ) - S, 0.0)

    total = jnp.sum(cls)
    pairs = jnp.sum(nv)
    out_ref[0, 0] = _WEIGHT * total / (pairs + _L1_REG * l1)


def kernel(pred, poses, ranks):
    B, _, H, W = pred.shape
    N = poses.shape[1]
    T = poses.shape[2]
    pred3 = pred.reshape(B, H * W)
    poses_flat = poses.reshape(B * N * T * 2)

    reward = _make_sc_gather(B, N, T, H, W)(pred3, poses_flat).reshape(B, N)

    out = pl.pallas_call(
        functools.partial(_tc_loss_body, B=B, N=N, n_elem=float(B * H * W)),
        out_shape=jax.ShapeDtypeStruct((1, 1), jnp.float32),
        out_specs=pl.BlockSpec(memory_space=pltpu.SMEM),
    )(pred.reshape(B * H, W), reward, ranks)
    return out.reshape(())


# speedup vs baseline: 7.3261x; 7.3261x over previous
import functools

import jax
import jax.numpy as jnp
from jax import lax
from jax.experimental import pallas as pl
from jax.experimental.pallas import tpu as pltpu
from jax.experimental.pallas import tpu_sc as plsc

_MAP_H = 64
_MAP_W = 128
_L1_REG = 0.1
_WEIGHT = 1.0


def _make_sc_gather(B, N, T, H, W):
    info = plsc.get_sparse_core_info()
    NC, NS, L = info.num_cores, info.num_subcores, info.num_lanes
    NW = NC * NS
    total_traj = B * N
    traj_per_w = total_traj // NW
    assert total_traj % NW == 0 and traj_per_w % 8 == 0
    n_full = T // L
    tail = T - n_full * L
    mesh = plsc.VectorSubcoreMesh(core_axis_name="c", subcore_axis_name="s")

    @functools.partial(
        pl.kernel,
        mesh=mesh,
        compiler_params=pltpu.CompilerParams(needs_layout_passes=False),
        out_type=jax.ShapeDtypeStruct((total_traj,), jnp.float32),
        scratch_types=[
            pltpu.VMEM((H * W,), jnp.float32),
            pltpu.VMEM((traj_per_w, T), jnp.int32),
            pltpu.VMEM((traj_per_w,), jnp.float32),
            pltpu.SemaphoreType.DMA,
        ],
    )
    def sc_gather(pred_hbm, poses_hbm, out_hbm,
                  pred_v, poses_v, reward_v, sem):
        wid = lax.axis_index("s") * NC + lax.axis_index("c")
        base = wid * traj_per_w
        b = base // N
        cp1 = pltpu.async_copy(pred_hbm.at[b], pred_v, sem)
        cp2 = pltpu.async_copy(poses_hbm.at[pl.ds(base, traj_per_w)], poses_v, sem)
        cp1.wait()
        cp2.wait()

        lane = lax.iota(jnp.int32, L)
        tail_mask = lane >= (L - tail)
        write_mask = lane == 0

        def flat_idx(pw):
            return (pw & 0xFFFF) * W + lax.shift_right_logical(pw, 16)

        def body(n, carry):
            acc = jnp.zeros((L,), jnp.float32)
            for j in range(n_full):
                pw = poses_v[n, pl.ds(j * L, L)]
                acc = acc + plsc.load_gather(pred_v, [flat_idx(pw)])
            if tail:
                pw = poses_v[n, pl.ds(T - L, L)]
                g = plsc.load_gather(pred_v, [flat_idx(pw)])
                acc = acc + jnp.where(tail_mask, g, 0.0)
            total = jnp.sum(acc)
            plsc.store_scatter(reward_v, [jnp.full((L,), n, jnp.int32)],
                               jnp.full((L,), total, jnp.float32),
                               mask=write_mask)
            return carry

        lax.fori_loop(0, traj_per_w, body, 0)
        pltpu.sync_copy(reward_v, out_hbm.at[pl.ds(base, traj_per_w)])

    return sc_gather


def _tc_loss_body(pred_ref, reward_ref, ranks_ref, out_ref, *, B, N, n_elem):
    predv = pred_ref[...]
    l1 = jnp.sum(jnp.abs(predv)) / n_elem

    ranks = ranks_ref[...]
    reward = reward_ref[...]
    pref = ranks == 0
    nprf = ranks > 0
    preff = pref.astype(jnp.float32)
    nprff = nprf.astype(jnp.float32)

    ii = lax.broadcasted_iota(jnp.int32, (N, N), 0)
    jj = lax.broadcasted_iota(jnp.int32, (N, N), 1)
    tri = (ii <= jj).astype(jnp.float32)
    pos_p = lax.dot(preff, tri).astype(jnp.int32) - 1
    pos_q = lax.dot(nprff, tri).astype(jnp.int32) - 1

    Np = jnp.sum(preff, axis=1, keepdims=True)
    Nn = jnp.sum(nprff, axis=1, keepdims=True)
    Npi = Np.astype(jnp.int32)
    Nni = Nn.astype(jnp.int32)

    def gcd_step(_, xy):
        x, y = xy
        cont = y > 0
        return (jnp.where(cont, y, x),
                jnp.where(cont, lax.rem(x, jnp.maximum(y, 1)), 0))

    gi, _ = lax.fori_loop(0, 12, gcd_step, (Npi, Nni))
    gsafe = jnp.maximum(gi, 1)
    rp = lax.rem(jnp.maximum(pos_p, 0), gsafe)
    rq = lax.rem(jnp.maximum(pos_q, 0), gsafe)

    A = reward[:, :, None]
    C = reward[:, None, :]
    m2 = jnp.maximum(A, C)
    nc = m2 + jnp.log1p(jnp.exp(-jnp.abs(A - C)))
    ap = A - nc
    cp = C - nc
    p1 = ap / (ap + cp + 1e-6)

    eq = rp.astype(jnp.float32)[:, :, None] == rq.astype(jnp.float32)[:, None, :]
    wf = (preff[:, :, None] * nprff[:, None, :]) * jnp.where(eq, 1.0, 0.0)
    w = wf > 0
    gf = gi.astype(jnp.float32)[:, :, None]

    nv = (Np * Nn)[:, 0]
    has = nv > 0
    S = jnp.sum(wf * p1 * gf, axis=(1, 2))
    M = jnp.max(jnp.where(w, p1, -jnp.inf), axis=(1, 2))
    Msafe = jnp.where(has, M, 0.0)
    Z = jnp.sum(wf * jnp.exp(p1 - Msafe[:, None, None]), axis=(1, 2)) * gf[:, 0, 0]
    logZ = jnp.where(has, jnp.log(jnp.maximum(Z, 1e-30)), 0.0)
    cls = jnp.where(has, nv * (Msafe + logZ) - S, 0.0)

    total = jnp.sum(cls)
    pairs = jnp.sum(nv)
    out_ref[0, 0] = _WEIGHT * total / (pairs + _L1_REG * l1)


def kernel(pred, poses, ranks):
    B, _, H, W = pred.shape
    N = poses.shape[1]
    T = poses.shape[2]
    pred3 = pred.reshape(B, H * W)
    poses_packed = lax.bitcast_convert_type(
        poses.astype(jnp.int16), jnp.int32).reshape(B * N, T)

    reward = _make_sc_gather(B, N, T, H, W)(pred3, poses_packed).reshape(B, N)

    out = pl.pallas_call(
        functools.partial(_tc_loss_body, B=B, N=N, n_elem=float(B * H * W)),
        out_shape=jax.ShapeDtypeStruct((1, 1), jnp.float32),
        out_specs=pl.BlockSpec(memory_space=pltpu.SMEM),
    )(pred.reshape(B * H, W), reward, ranks)
    return out.reshape(())


# speedup vs baseline: 7.3378x; 1.0016x over previous
import functools

import jax
import jax.numpy as jnp
from jax import lax
from jax.experimental import pallas as pl
from jax.experimental.pallas import tpu as pltpu
from jax.experimental.pallas import tpu_sc as plsc

_MAP_H = 64
_MAP_W = 128
_L1_REG = 0.1
_WEIGHT = 1.0


def _make_sc_gather(B, N, T, H, W):
    info = plsc.get_sparse_core_info()
    NC, NS, L = info.num_cores, info.num_subcores, info.num_lanes
    NW = NC * NS
    total_traj = B * N
    traj_per_w = total_traj // NW
    assert total_traj % NW == 0 and traj_per_w % 8 == 0
    n_full = T // L
    tail = T - n_full * L
    mesh = plsc.VectorSubcoreMesh(core_axis_name="c", subcore_axis_name="s")

    @functools.partial(
        pl.kernel,
        mesh=mesh,
        compiler_params=pltpu.CompilerParams(needs_layout_passes=False),
        out_type=(jax.ShapeDtypeStruct((total_traj,), jnp.float32),
                  jax.ShapeDtypeStruct((NW * L,), jnp.float32)),
        scratch_types=[
            pltpu.VMEM((H * W,), jnp.float32),
            pltpu.VMEM((traj_per_w, T), jnp.int32),
            pltpu.VMEM((traj_per_w,), jnp.float32),
            pltpu.VMEM((L,), jnp.float32),
            pltpu.SemaphoreType.DMA,
        ],
    )
    def sc_gather(pred_hbm, poses_hbm, out_hbm, l1_hbm,
                  pred_v, poses_v, reward_v, l1_v, sem):
        wid = lax.axis_index("s") * NC + lax.axis_index("c")
        base = wid * traj_per_w
        b = base // N
        cp1 = pltpu.async_copy(pred_hbm.at[b], pred_v, sem)
        cp2 = pltpu.async_copy(poses_hbm.at[pl.ds(base, traj_per_w)], poses_v, sem)
        cp1.wait()
        cp2.wait()

        lane = lax.iota(jnp.int32, L)
        tail_mask = lane >= (L - tail)
        write_mask = lane == 0

        def flat_idx(pw):
            return (pw & 0xFFFF) * W + lax.shift_right_logical(pw, 16)

        def body(n, carry):
            acc = jnp.zeros((L,), jnp.float32)
            for j in range(n_full):
                pw = poses_v[n, pl.ds(j * L, L)]
                acc = acc + plsc.load_gather(pred_v, [flat_idx(pw)])
            if tail:
                pw = poses_v[n, pl.ds(T - L, L)]
                g = plsc.load_gather(pred_v, [flat_idx(pw)])
                acc = acc + jnp.where(tail_mask, g, 0.0)
            total = jnp.sum(acc)
            plsc.store_scatter(reward_v, [jnp.full((L,), n, jnp.int32)],
                               jnp.full((L,), total, jnp.float32),
                               mask=write_mask)
            return carry

        lax.fori_loop(0, traj_per_w, body, 0)
        pltpu.sync_copy(reward_v, out_hbm.at[pl.ds(base, traj_per_w)])

        half = (H * W) // 2
        hoff = (wid % 2) * half

        def l1_body(i, acc):
            o = hoff + i * (8 * L)
            for u in range(8):
                acc = acc + jnp.abs(pred_v[pl.ds(o + u * L, L)])
            return acc

        acc16 = lax.fori_loop(0, half // (8 * L), l1_body,
                              jnp.zeros((L,), jnp.float32))
        l1_v[...] = jnp.zeros((L,), jnp.float32)
        plsc.store_scatter(l1_v, [jnp.full((L,), 0, jnp.int32)],
                           jnp.full((L,), jnp.sum(acc16), jnp.float32),
                           mask=write_mask)
        pltpu.sync_copy(l1_v, l1_hbm.at[pl.ds(wid * L, L)])

    return sc_gather


def _tc_loss_body(l1p_ref, reward_ref, ranks_ref, out_ref, *, B, N, n_elem):
    l1 = jnp.sum(l1p_ref[...]) / n_elem

    ranks = ranks_ref[...]
    reward = reward_ref[...]
    pref = ranks == 0
    nprf = ranks > 0
    preff = pref.astype(jnp.float32)
    nprff = nprf.astype(jnp.float32)

    ii = lax.broadcasted_iota(jnp.int32, (N, N), 0)
    jj = lax.broadcasted_iota(jnp.int32, (N, N), 1)
    tri = (ii <= jj).astype(jnp.float32)
    pos_p = lax.dot(preff, tri).astype(jnp.int32) - 1
    pos_q = lax.dot(nprff, tri).astype(jnp.int32) - 1

    Np = jnp.sum(preff, axis=1, keepdims=True)
    Nn = jnp.sum(nprff, axis=1, keepdims=True)
    Npi = Np.astype(jnp.int32)
    Nni = Nn.astype(jnp.int32)

    def gcd_step(_, xy):
        x, y = xy
        cont = y > 0
        return (jnp.where(cont, y, x),
                jnp.where(cont, lax.rem(x, jnp.maximum(y, 1)), 0))

    gi, _ = lax.fori_loop(0, 12, gcd_step, (Npi, Nni))
    gsafe = jnp.maximum(gi, 1)
    rp = lax.rem(jnp.maximum(pos_p, 0), gsafe)
    rq = lax.rem(jnp.maximum(pos_q, 0), gsafe)

    A = reward[:, :, None]
    C = reward[:, None, :]
    m2 = jnp.maximum(A, C)
    nc = m2 + jnp.log1p(jnp.exp(-jnp.abs(A - C)))
    ap = A - nc
    cp = C - nc
    p1 = ap / (ap + cp + 1e-6)

    eq = rp.astype(jnp.float32)[:, :, None] == rq.astype(jnp.float32)[:, None, :]
    wf = (preff[:, :, None] * nprff[:, None, :]) * jnp.where(eq, 1.0, 0.0)
    w = wf > 0
    gf = gi.astype(jnp.float32)[:, :, None]

    nv = (Np * Nn)[:, 0]
    has = nv > 0
    S = jnp.sum(wf * p1 * gf, axis=(1, 2))
    M = jnp.max(jnp.where(w, p1, -jnp.inf), axis=(1, 2))
    Msafe = jnp.where(has, M, 0.0)
    Z = jnp.sum(wf * jnp.exp(p1 - Msafe[:, None, None]), axis=(1, 2)) * gf[:, 0, 0]
    logZ = jnp.where(has, jnp.log(jnp.maximum(Z, 1e-30)), 0.0)
    cls = jnp.where(has, nv * (Msafe + logZ) - S, 0.0)

    total = jnp.sum(cls)
    pairs = jnp.sum(nv)
    out_ref[0, 0] = _WEIGHT * total / (pairs + _L1_REG * l1)


def kernel(pred, poses, ranks):
    B, _, H, W = pred.shape
    N = poses.shape[1]
    T = poses.shape[2]
    pred3 = pred.reshape(B, H * W)
    poses_packed = lax.bitcast_convert_type(
        poses.astype(jnp.int16), jnp.int32).reshape(B * N, T)

    reward_flat, l1_parts = _make_sc_gather(B, N, T, H, W)(pred3, poses_packed)
    reward = reward_flat.reshape(B, N)

    out = pl.pallas_call(
        functools.partial(_tc_loss_body, B=B, N=N, n_elem=float(B * H * W)),
        out_shape=jax.ShapeDtypeStruct((1, 1), jnp.float32),
        out_specs=pl.BlockSpec(memory_space=pltpu.SMEM),
    )(l1_parts, reward, ranks)
    return out.reshape(())


# speedup vs baseline: 7.6006x; 1.0358x over previous
import functools

import jax
import jax.numpy as jnp
from jax import lax
from jax.experimental import pallas as pl
from jax.experimental.pallas import tpu as pltpu
from jax.experimental.pallas import tpu_sc as plsc

_MAP_H = 64
_MAP_W = 128
_L1_REG = 0.1
_WEIGHT = 1.0


def _make_sc_gather(B, N, T, H, W):
    info = plsc.get_sparse_core_info()
    NC, NS, L = info.num_cores, info.num_subcores, info.num_lanes
    NW = NC * NS
    total_traj = B * N
    traj_per_w = total_traj // NW
    assert total_traj % NW == 0 and traj_per_w % 8 == 0
    n_full = T // L
    tail = T - n_full * L
    mesh = plsc.VectorSubcoreMesh(core_axis_name="c", subcore_axis_name="s")

    @functools.partial(
        pl.kernel,
        mesh=mesh,
        compiler_params=pltpu.CompilerParams(needs_layout_passes=False),
        out_type=(jax.ShapeDtypeStruct((total_traj,), jnp.float32),
                  jax.ShapeDtypeStruct((NW * L,), jnp.float32)),
        scratch_types=[
            pltpu.VMEM((H, W), jnp.float32),
            pltpu.VMEM((traj_per_w, T), jnp.int32),
            pltpu.VMEM((traj_per_w,), jnp.float32),
            pltpu.VMEM((L,), jnp.float32),
            pltpu.SemaphoreType.DMA,
        ],
    )
    def sc_gather(pred_hbm, poses_hbm, out_hbm, l1_hbm,
                  pred_v, poses_v, reward_v, l1_v, sem):
        wid = lax.axis_index("s") * NC + lax.axis_index("c")
        base = wid * traj_per_w
        b = base // N
        cp1 = pltpu.async_copy(pred_hbm.at[b], pred_v, sem)
        cp2 = pltpu.async_copy(poses_hbm.at[pl.ds(base, traj_per_w)], poses_v, sem)
        cp1.wait()
        cp2.wait()

        lane = lax.iota(jnp.int32, L)
        tail_mask = lane >= (L - tail)
        write_mask = lane == 0

        def rc(pw):
            return pw & 0xFFFF, lax.shift_right_logical(pw, 16)

        def body(n, carry):
            acc = jnp.zeros((L,), jnp.float32)
            for j in range(n_full):
                r, c = rc(poses_v[n, pl.ds(j * L, L)])
                acc = acc + plsc.load_gather(pred_v, [r, c])
            if tail:
                r, c = rc(poses_v[n, pl.ds(T - L, L)])
                g = plsc.load_gather(pred_v, [r, c])
                acc = acc + jnp.where(tail_mask, g, 0.0)
            total = jnp.sum(acc)
            plsc.store_scatter(reward_v, [jnp.full((L,), n, jnp.int32)],
                               jnp.full((L,), total, jnp.float32),
                               mask=write_mask)
            return carry

        lax.fori_loop(0, traj_per_w, body, 0)
        pltpu.sync_copy(reward_v, out_hbm.at[pl.ds(base, traj_per_w)])

        half = (H * W) // 2
        hoff = (wid % 2) * half

        rows_per_it = (8 * L) // W
        def l1_body(i, acc):
            o = (hoff + i * (8 * L)) // W
            for u in range(8 * L // W):
                acc16 = jnp.zeros((L,), jnp.float32)
                for v in range(W // L):
                    acc16 = acc16 + jnp.abs(pred_v[o + u, pl.ds(v * L, L)])
                acc = acc + acc16
            return acc

        acc16 = lax.fori_loop(0, half // (8 * L), l1_body,
                              jnp.zeros((L,), jnp.float32))
        l1_v[...] = jnp.zeros((L,), jnp.float32)
        plsc.store_scatter(l1_v, [jnp.full((L,), 0, jnp.int32)],
                           jnp.full((L,), jnp.sum(acc16), jnp.float32),
                           mask=write_mask)
        pltpu.sync_copy(l1_v, l1_hbm.at[pl.ds(wid * L, L)])

    return sc_gather


def _tc_loss_body(l1p_ref, reward_ref, ranks_ref, out_ref, *, B, N, n_elem):
    l1 = jnp.sum(l1p_ref[...]) / n_elem

    ranks = ranks_ref[...]
    reward = reward_ref[...]
    pref = ranks == 0
    nprf = ranks > 0
    preff = pref.astype(jnp.float32)
    nprff = nprf.astype(jnp.float32)

    ii = lax.broadcasted_iota(jnp.int32, (N, N), 0)
    jj = lax.broadcasted_iota(jnp.int32, (N, N), 1)
    tri = (ii <= jj).astype(jnp.float32)
    pos_p = lax.dot(preff, tri).astype(jnp.int32) - 1
    pos_q = lax.dot(nprff, tri).astype(jnp.int32) - 1

    Np = jnp.sum(preff, axis=1, keepdims=True)
    Nn = jnp.sum(nprff, axis=1, keepdims=True)
    Npi = Np.astype(jnp.int32)
    Nni = Nn.astype(jnp.int32)

    def gcd_step(_, xy):
        x, y = xy
        cont = y > 0
        return (jnp.where(cont, y, x),
                jnp.where(cont, lax.rem(x, jnp.maximum(y, 1)), 0))

    gi, _ = lax.fori_loop(0, 12, gcd_step, (Npi, Nni))
    gsafe = jnp.maximum(gi, 1)
    rp = lax.rem(jnp.maximum(pos_p, 0), gsafe)
    rq = lax.rem(jnp.maximum(pos_q, 0), gsafe)

    A = reward[:, :, None]
    C = reward[:, None, :]
    m2 = jnp.maximum(A, C)
    nc = m2 + jnp.log1p(jnp.exp(-jnp.abs(A - C)))
    ap = A - nc
    cp = C - nc
    p1 = ap / (ap + cp + 1e-6)

    eq = rp.astype(jnp.float32)[:, :, None] == rq.astype(jnp.float32)[:, None, :]
    wf = (preff[:, :, None] * nprff[:, None, :]) * jnp.where(eq, 1.0, 0.0)
    w = wf > 0
    gf = gi.astype(jnp.float32)[:, :, None]

    nv = (Np * Nn)[:, 0]
    has = nv > 0
    S = jnp.sum(wf * p1 * gf, axis=(1, 2))
    M = jnp.max(jnp.where(w, p1, -jnp.inf), axis=(1, 2))
    Msafe = jnp.where(has, M, 0.0)
    Z = jnp.sum(wf * jnp.exp(p1 - Msafe[:, None, None]), axis=(1, 2)) * gf[:, 0, 0]
    logZ = jnp.where(has, jnp.log(jnp.maximum(Z, 1e-30)), 0.0)
    cls = jnp.where(has, nv * (Msafe + logZ) - S, 0.0)

    total = jnp.sum(cls)
    pairs = jnp.sum(nv)
    out_ref[0, 0] = _WEIGHT * total / (pairs + _L1_REG * l1)


def kernel(pred, poses, ranks):
    B, _, H, W = pred.shape
    N = poses.shape[1]
    T = poses.shape[2]
    pred3 = pred.reshape(B, H, W)
    poses_packed = lax.bitcast_convert_type(
        poses.astype(jnp.int16), jnp.int32).reshape(B * N, T)

    reward_flat, l1_parts = _make_sc_gather(B, N, T, H, W)(pred3, poses_packed)
    reward = reward_flat.reshape(B, N)

    out = pl.pallas_call(
        functools.partial(_tc_loss_body, B=B, N=N, n_elem=float(B * H * W)),
        out_shape=jax.ShapeDtypeStruct((1, 1), jnp.float32),
        out_specs=pl.BlockSpec(memory_space=pltpu.SMEM),
    )(l1_parts, reward, ranks)
    return out.reshape(())


# speedup vs baseline: 7.7370x; 1.0179x over previous
import functools

import jax
import jax.numpy as jnp
from jax import lax
from jax.experimental import pallas as pl
from jax.experimental.pallas import tpu as pltpu
from jax.experimental.pallas import tpu_sc as plsc

_MAP_H = 64
_MAP_W = 128
_L1_REG = 0.1
_WEIGHT = 1.0


def _make_sc_gather(B, N, T, H, W):
    info = plsc.get_sparse_core_info()
    NC, NS, L = info.num_cores, info.num_subcores, info.num_lanes
    NW = NC * NS
    total_traj = B * N
    traj_per_w = total_traj // NW
    assert total_traj % NW == 0 and traj_per_w % 8 == 0
    n_full = T // L
    tail = T - n_full * L
    mesh = plsc.VectorSubcoreMesh(core_axis_name="c", subcore_axis_name="s")

    @functools.partial(
        pl.kernel,
        mesh=mesh,
        compiler_params=pltpu.CompilerParams(needs_layout_passes=False),
        out_type=(jax.ShapeDtypeStruct((total_traj,), jnp.float32),
                  jax.ShapeDtypeStruct((NW * L,), jnp.float32)),
        scratch_types=[
            pltpu.VMEM((H, W), jnp.float32),
            pltpu.VMEM((traj_per_w, T), jnp.int32),
            pltpu.VMEM((traj_per_w,), jnp.float32),
            pltpu.VMEM((L,), jnp.float32),
            pltpu.SemaphoreType.DMA,
        ],
    )
    def sc_gather(pred_hbm, poses_hbm, out_hbm, l1_hbm,
                  pred_v, poses_v, reward_v, l1_v, sem):
        wid = lax.axis_index("s") * NC + lax.axis_index("c")
        base = wid * traj_per_w
        b = base // N
        cp1 = pltpu.async_copy(pred_hbm.at[b], pred_v, sem)
        cp2 = pltpu.async_copy(poses_hbm.at[pl.ds(base, traj_per_w)], poses_v, sem)
        cp1.wait()
        cp2.wait()

        lane = lax.iota(jnp.int32, L)
        tail_mask = lane >= (L - tail)
        write_mask = lane == 0

        def rc(pw):
            return pw & 0xFFFF, lax.shift_right_logical(pw, 16)

        def one_traj(n):
            acc = jnp.zeros((L,), jnp.float32)
            for j in range(n_full):
                r, c = rc(poses_v[n, pl.ds(j * L, L)])
                acc = acc + plsc.load_gather(pred_v, [r, c])
            if tail:
                r, c = rc(poses_v[n, pl.ds(T - L, L)])
                g = plsc.load_gather(pred_v, [r, c])
                acc = acc + jnp.where(tail_mask, g, 0.0)
            return jnp.sum(acc)

        def body(m, carry):
            n = m * 2
            t0 = one_traj(n)
            t1 = one_traj(n + 1)
            plsc.store_scatter(reward_v, [jnp.full((L,), n, jnp.int32)],
                               jnp.full((L,), t0, jnp.float32),
                               mask=write_mask)
            plsc.store_scatter(reward_v, [jnp.full((L,), n + 1, jnp.int32)],
                               jnp.full((L,), t1, jnp.float32),
                               mask=write_mask)
            return carry

        lax.fori_loop(0, traj_per_w // 2, body, 0)
        pltpu.sync_copy(reward_v, out_hbm.at[pl.ds(base, traj_per_w)])

        half = (H * W) // 2
        hoff = (wid % 2) * half

        rows_per_it = (8 * L) // W
        def l1_body(i, acc):
            o = (hoff + i * (8 * L)) // W
            for u in range(8 * L // W):
                acc16 = jnp.zeros((L,), jnp.float32)
                for v in range(W // L):
                    acc16 = acc16 + jnp.abs(pred_v[o + u, pl.ds(v * L, L)])
                acc = acc + acc16
            return acc

        acc16 = lax.fori_loop(0, half // (8 * L), l1_body,
                              jnp.zeros((L,), jnp.float32))
        l1_v[...] = jnp.zeros((L,), jnp.float32)
        plsc.store_scatter(l1_v, [jnp.full((L,), 0, jnp.int32)],
                           jnp.full((L,), jnp.sum(acc16), jnp.float32),
                           mask=write_mask)
        pltpu.sync_copy(l1_v, l1_hbm.at[pl.ds(wid * L, L)])

    return sc_gather


def _tc_loss_body(l1p_ref, reward_ref, ranks_ref, out_ref, *, B, N, n_elem):
    l1 = jnp.sum(l1p_ref[...]) / n_elem

    ranks = ranks_ref[...]
    reward = reward_ref[...]
    pref = ranks == 0
    nprf = ranks > 0
    preff = pref.astype(jnp.float32)
    nprff = nprf.astype(jnp.float32)

    ii = lax.broadcasted_iota(jnp.int32, (N, N), 0)
    jj = lax.broadcasted_iota(jnp.int32, (N, N), 1)
    tri = (ii <= jj).astype(jnp.float32)
    pos_p = lax.dot(preff, tri).astype(jnp.int32) - 1
    pos_q = lax.dot(nprff, tri).astype(jnp.int32) - 1

    Np = jnp.sum(preff, axis=1, keepdims=True)
    Nn = jnp.sum(nprff, axis=1, keepdims=True)
    Npi = Np.astype(jnp.int32)
    Nni = Nn.astype(jnp.int32)

    def gcd_step(_, xy):
        x, y = xy
        cont = y > 0
        return (jnp.where(cont, y, x),
                jnp.where(cont, lax.rem(x, jnp.maximum(y, 1)), 0))

    gi, _ = lax.fori_loop(0, 12, gcd_step, (Npi, Nni))
    gsafe = jnp.maximum(gi, 1)
    rp = lax.rem(jnp.maximum(pos_p, 0), gsafe)
    rq = lax.rem(jnp.maximum(pos_q, 0), gsafe)

    A = reward[:, :, None]
    C = reward[:, None, :]
    m2 = jnp.maximum(A, C)
    nc = m2 + jnp.log1p(jnp.exp(-jnp.abs(A - C)))
    ap = A - nc
    cp = C - nc
    p1 = ap / (ap + cp + 1e-6)

    eq = rp.astype(jnp.float32)[:, :, None] == rq.astype(jnp.float32)[:, None, :]
    wf = (preff[:, :, None] * nprff[:, None, :]) * jnp.where(eq, 1.0, 0.0)
    w = wf > 0
    gf = gi.astype(jnp.float32)[:, :, None]

    nv = (Np * Nn)[:, 0]
    has = nv > 0
    S = jnp.sum(wf * p1 * gf, axis=(1, 2))
    M = jnp.max(jnp.where(w, p1, -jnp.inf), axis=(1, 2))
    Msafe = jnp.where(has, M, 0.0)
    Z = jnp.sum(wf * jnp.exp(p1 - Msafe[:, None, None]), axis=(1, 2)) * gf[:, 0, 0]
    logZ = jnp.where(has, jnp.log(jnp.maximum(Z, 1e-30)), 0.0)
    cls = jnp.where(has, nv * (Msafe + logZ) - S, 0.0)

    total = jnp.sum(cls)
    pairs = jnp.sum(nv)
    out_ref[0, 0] = _WEIGHT * total / (pairs + _L1_REG * l1)


def kernel(pred, poses, ranks):
    B, _, H, W = pred.shape
    N = poses.shape[1]
    T = poses.shape[2]
    pred3 = pred.reshape(B, H, W)
    poses_packed = lax.bitcast_convert_type(
        poses.astype(jnp.int16), jnp.int32).reshape(B * N, T)

    reward_flat, l1_parts = _make_sc_gather(B, N, T, H, W)(pred3, poses_packed)
    reward = reward_flat.reshape(B, N)

    out = pl.pallas_call(
        functools.partial(_tc_loss_body, B=B, N=N, n_elem=float(B * H * W)),
        out_shape=jax.ShapeDtypeStruct((1, 1), jnp.float32),
        out_specs=pl.BlockSpec(memory_space=pltpu.SMEM),
    )(l1_parts, reward, ranks)
    return out.reshape(())
